# baseline (speedup 1.0000x reference)
import functools

import jax
import jax.numpy as jnp
from jax import lax
from jax.experimental import pallas as pl
from jax.experimental.pallas import tpu as pltpu

_B, _N, _C = 8, 5000, 21
_NP = 5120
_K = 500
_KP = 512
_THR = 0.4
_TEMP = 0.1
_TAU = 0.05
_NEG = -1.0e30


def _fiota(shape, dim):
    return lax.broadcasted_iota(jnp.int32, shape, dim).astype(jnp.float32)

def _body(preds_ref, predsT_ref, boxes_ref, trueT_ref, out_ref,
          colf, srow, rrow, self_ref, i512):
    img = pl.program_id(0)
    f32 = jnp.float32

    @pl.when(img == 0)
    def _init():
        out_ref[...] = jnp.zeros((1, 1), f32)
        ii = _fiota((_KP, _KP), 0)
        jj = _fiota((_KP, _KP), 1)
        i512[...] = jnp.where(ii == jj, 1.0, 0.0).astype(f32)
        padlane = lax.broadcasted_iota(jnp.int32, (_NP - _N, 8), 1)
        colf[_N:_NP, :] = jnp.where(padlane == 0, _NEG, 0.0).astype(f32)
        srow[0:1, _N:_NP] = jnp.full((1, _NP - _N), _NEG, f32)

    p = preds_ref[0]
    lane = lax.broadcasted_iota(jnp.int32, (_N, _C), 1)
    s_col = jnp.max(jnp.where(lane >= 1, p, _NEG), axis=1, keepdims=True)
    colf[0:_N, 0:1] = s_col
    colf[0:_N, 1:5] = boxes_ref[0]
    tt = trueT_ref[...]
    lane8 = lax.broadcasted_iota(jnp.int32, (_N, _B), 1)
    t_col = jnp.sum(jnp.where(lane8 == img, tt, 0.0), axis=1, keepdims=True)
    colf[0:_N, 5:6] = jnp.where(t_col > 0.5, 1.0, 0.0).astype(f32)

    pT = predsT_ref[0]
    rl = lax.broadcasted_iota(jnp.int32, (_C, _N), 0)
    srow[0:1, 0:_N] = jnp.max(jnp.where(rl >= 1, pT, _NEG), axis=0,
                              keepdims=True)

    def rank_chunk(ic, _):
        si = srow[0:1, pl.ds(ic * _KP, _KP)]
        i_idx = (_fiota((1, _KP), 1)
                 + ic.astype(f32) * float(_KP))

        def jstep(jc, acc):
            sj = colf[pl.ds(jc * 256, 256), 0:1]
            j_idx = (_fiota((256, 1), 0)
                     + jc.astype(f32) * 256.0)
            g = (sj > si) | ((sj == si) & (j_idx < i_idx))
            return acc + jnp.sum(g.astype(f32), axis=0, keepdims=True)

        acc = lax.fori_loop(0, _NP // 256, jstep, jnp.zeros((1, _KP), f32))
        rrow[0:1, pl.ds(ic * _KP, _KP)] = acc
        return 0

    lax.fori_loop(0, _NP // _KP, rank_chunk, 0)

    self_ref[...] = jnp.zeros((_KP, 8), f32)

    def sel_chunk(jc, _):
        r_j = rrow[0:1, pl.ds(jc * _KP, _KP)]
        keep = r_j < float(_K)
        feats = colf[pl.ds(jc * _KP, _KP), :]
        for rb in range(_KP // 128):
            rr = (_fiota((128, _KP), 0)
                  + float(rb * 128))
            ptile = jnp.where((rr == r_j) & keep, 1.0, 0.0).astype(f32)
            self_ref[pl.ds(rb * 128, 128), :] += jnp.dot(
                ptile, feats, preferred_element_type=f32)
        return 0

    lax.fori_loop(0, _NP // _KP, sel_chunk, 0)

    selT = lax.dot_general(self_ref[...], i512[...],
                           (((0,), (0,)), ((), ())),
                           preferred_element_type=f32)

    s_row = selT[0:1, :]
    x1r, y1r = selT[1:2, :], selT[2:3, :]
    x2r, y2r = selT[3:4, :], selT[4:5, :]
    area_r = (x2r - x1r) * (y2r - y1r)
    jjl = _fiota((1, _KP), 1)

    for rb in range(_KP // 128):
        rows = pl.ds(rb * 128, 128)
        x1c, y1c = self_ref[rows, 1:2], self_ref[rows, 2:3]
        x2c, y2c = self_ref[rows, 3:4], self_ref[rows, 4:5]
        area_c = (x2c - x1c) * (y2c - y1c)
        iw = jnp.maximum(jnp.minimum(x2c, x2r) - jnp.maximum(x1c, x1r), 0.0)
        ih = jnp.maximum(jnp.minimum(y2c, y2r) - jnp.maximum(y1c, y1r), 0.0)
        inter = iw * ih
        iou = inter / (area_c + area_r - inter + 1e-9)
        lg = jnp.log(jax.nn.sigmoid((_THR - iou) / _TEMP) + 1e-12)
        iic = _fiota((128, 1), 0) + float(rb * 128)
        lk = jnp.sum(jnp.where(jjl < iic, lg, 0.0), axis=1, keepdims=True)
        self_ref[rows, 6:7] = self_ref[rows, 0:1] * jnp.exp(lk)

    nsr = lax.dot_general(self_ref[:, 6:7], i512[...],
                          (((0,), (0,)), ((), ())),
                          preferred_element_type=f32)
    tr = selT[5:6, :]
    valid = jjl < float(_K)
    n_pos = jnp.sum(self_ref[:, 5:6])
    acc_ap = jnp.zeros((1, 1), f32)
    for rb in range(_KP // 128):
        rows = pl.ds(rb * 128, 128)
        s_i = self_ref[rows, 6:7]
        sg = jax.nn.sigmoid((nsr - s_i) / _TAU)
        iic = _fiota((128, 1), 0) + float(rb * 128)
        w = jnp.where((jjl != iic) & valid, sg, 0.0)
        rank_all = 1.0 + jnp.sum(w, axis=1, keepdims=True)
        rank_pos = 1.0 + jnp.sum(w * tr, axis=1, keepdims=True)
        prec = rank_pos / rank_all
        acc_ap += jnp.sum(prec * self_ref[rows, 5:6], axis=0,
                          keepdims=True).reshape(1, 1)
    ap = acc_ap / jnp.maximum(n_pos, 1.0)
    loss = jnp.where(n_pos > 0.0, 1.0 - ap, jnp.zeros((1, 1), f32))
    out_ref[...] += loss / float(_B)


@functools.partial(jax.jit, static_argnames=("interpret",))
def _run(preds, pred, true, interpret=False):
    predsT = jnp.swapaxes(preds, 1, 2)
    trueT = jnp.swapaxes(true, 0, 1)
    f32 = jnp.float32
    out = pl.pallas_call(
        _body,
        grid=(_B,),
        in_specs=[
            pl.BlockSpec((1, _N, _C), lambda i: (i, 0, 0)),
            pl.BlockSpec((1, _C, _N), lambda i: (i, 0, 0)),
            pl.BlockSpec((1, _N, 4), lambda i: (i, 0, 0)),
            pl.BlockSpec((_N, _B), lambda i: (0, 0)),
        ],
        out_specs=pl.BlockSpec((1, 1), lambda i: (0, 0)),
        out_shape=jax.ShapeDtypeStruct((1, 1), f32),
        scratch_shapes=[
            pltpu.VMEM((_NP, 8), f32),
            pltpu.VMEM((1, _NP), f32),
            pltpu.VMEM((1, _NP), f32),
            pltpu.VMEM((_KP, 8), f32),
            pltpu.VMEM((_KP, _KP), f32),
        ],
        compiler_params=pltpu.CompilerParams(
            dimension_semantics=("arbitrary",)),
        interpret=interpret,
    )(preds, predsT, pred, trueT)
    return out[0, 0]


def kernel(preds, pred, true):
    return _run(preds, pred, true)


# speedup vs baseline: 2.3037x; 2.3037x over previous
import functools

import jax
import jax.numpy as jnp
from jax import lax
from jax.experimental import pallas as pl
from jax.experimental.pallas import tpu as pltpu
from jax.experimental.pallas import tpu_sc as plsc

_B, _N, _C = 8, 5000, 21
_NS = 5008
_NF = 5016
_K = 500
_KP = 512
_THR = 0.4
_TEMP = 0.1
_TAU = 0.05
_NEG = -1.0e30
_NV = _NS // 16
_FW = 16
_PACK = 128 // _FW


def _fiota(shape, dim):
    return lax.broadcasted_iota(jnp.int32, shape, dim).astype(jnp.float32)


def _prep_body(preds_ref, predsT_ref, boxes_ref, trueT_ref,
               scores_ref, feats_ref):
    img = pl.program_id(0)
    f32 = jnp.float32

    pT = predsT_ref[0]
    rl = lax.broadcasted_iota(jnp.int32, (_C, _N), 0)
    srow = jnp.max(jnp.where(rl >= 1, pT, _NEG), axis=0, keepdims=True)
    scores_ref[0, 0:1, 0:_N] = srow
    scores_ref[0, 0:1, _N:_NS] = jnp.full((1, _NS - _N), _NEG, f32)

    p = preds_ref[0]
    lane = lax.broadcasted_iota(jnp.int32, (_N, _C), 1)
    s_col = jnp.max(jnp.where(lane >= 1, p, _NEG), axis=1, keepdims=True)

    tt = trueT_ref[...]
    lane8 = lax.broadcasted_iota(jnp.int32, (_N, _B), 1)
    t_col = jnp.sum(jnp.where(lane8 == img, tt, 0.0), axis=1, keepdims=True)

    feats_ref[0, 0:_N, 0:1] = s_col
    feats_ref[0, 0:_N, 1:5] = boxes_ref[0]
    feats_ref[0, 0:_N, 5:6] = jnp.where(t_col > 0.5, 1.0, 0.0).astype(f32)
    feats_ref[0, :, 6:7] = _fiota((_NF, 1), 0)
    feats_ref[0, :, 7:_FW] = jnp.zeros((_NF, _FW - 7), f32)
    feats_ref[0, _N:_NF, 0:6] = jnp.where(
        lax.broadcasted_iota(jnp.int32, (_NF - _N, 6), 1) == 0,
        _NEG, 0.0).astype(f32)


def _tc_prep(preds, predsT, boxes, trueT):
    f32 = jnp.float32
    return pl.pallas_call(
        _prep_body,
        grid=(_B,),
        in_specs=[
            pl.BlockSpec((1, _N, _C), lambda i: (i, 0, 0)),
            pl.BlockSpec((1, _C, _N), lambda i: (i, 0, 0)),
            pl.BlockSpec((1, _N, 4), lambda i: (i, 0, 0)),
            pl.BlockSpec((_N, _B), lambda i: (0, 0)),
        ],
        out_specs=[
            pl.BlockSpec((1, 1, _NS), lambda i: (i, 0, 0)),
            pl.BlockSpec((1, _NF, _FW), lambda i: (i, 0, 0)),
        ],
        out_shape=[
            jax.ShapeDtypeStruct((_B, 1, _NS), f32),
            jax.ShapeDtypeStruct((_B, _NF, _FW), f32),
        ],
        compiler_params=pltpu.CompilerParams(
            dimension_semantics=("arbitrary",)),
    )(preds, predsT, boxes, trueT)


def _sc_select(scores, feats_packed):
    mesh = plsc.VectorSubcoreMesh(core_axis_name="c", subcore_axis_name="s")
    i32, f32 = jnp.int32, jnp.float32
    rows_per_img = (_NF * _FW) // 128

    @functools.partial(
        pl.kernel,
        mesh=mesh,
        out_type=jax.ShapeDtypeStruct((_B, _KP * _FW), f32),
        scratch_types=[
            pltpu.VMEM((_NS,), f32),
            pltpu.VMEM((4, 128), i32),
            pltpu.VMEM((_KP,), i32),
            pltpu.VMEM((_KP, 128), f32),
            pltpu.VMEM((_KP * _FW,), f32),
            pltpu.SMEM((_KP,), i32),
            pltpu.SemaphoreType.DMA,
        ],
    )
    def sel_kernel(scores_hbm, feats_hbm, out_hbm,
                   sv, rowv, idxv, gat, ov, sm, sem):
        wid = lax.axis_index("s") * 2 + lax.axis_index("c")

        @pl.when(wid < _B)
        def _work():
            img = wid
            pltpu.sync_copy(scores_hbm.at[img, 0], sv)

            iota16 = lax.broadcasted_iota(i32, (16,), 0)
            dn = lax.GatherDimensionNumbers(
                offset_dims=(), collapsed_slice_dims=(0,),
                start_index_map=(0,))

            def lanesum(x):
                for sh in (8, 4, 2, 1):
                    idxp = (iota16 + sh) % 16
                    x = x + lax.gather(
                        x, idxp[:, None], dn, (1,),
                        mode=lax.GatherScatterMode.PROMISE_IN_BOUNDS)
                return x[0]

            def cnt_gt(thrf):
                def body(v, acc):
                    s = sv[pl.ds(v * 16, 16)]
                    return acc + jnp.where(s > thrf, 1, 0)
                accv = lax.fori_loop(0, _NV, body, jnp.zeros((16,), i32))
                return lanesum(accv)

            def bs(_, lohi):
                lo, hi = lohi
                mid = (lo + hi) // 2
                midf = lax.bitcast_convert_type(mid, f32)
                big = cnt_gt(midf) >= _K
                return (jnp.where(big, mid, lo), jnp.where(big, hi, mid))
            _, hi = lax.fori_loop(
                0, 31, bs, (jnp.int32(-1), jnp.int32(0x3F800000)))
            ustarf = lax.bitcast_convert_type(hi, f32)

            for j in range(_KP - _K):
                sm[_K + j] = jnp.int32(_N + j)

            def sweep(pred_fn, capped):
                def body(v, cnt):
                    s = sv[pl.ds(v * 16, 16)]
                    mi = jnp.where(pred_fn(s), 1, 0)
                    vecsum = lanesum(mi)

                    def write(c):
                        for l in range(16):
                            take = mi[l]
                            if capped:
                                take = jnp.where(c < _K, take, 0)
                            cw = jnp.minimum(c, _KP - 1)
                            old = sm[cw]
                            sm[cw] = jnp.where(take > 0, v * 16 + l, old)
                            c = c + take
                        return c
                    return lax.cond(vecsum > 0, write, lambda c: c, cnt)
                return body

            cnt = lax.fori_loop(0, _NV, sweep(lambda s: s > ustarf, False),
                                jnp.int32(0))
            lax.fori_loop(0, _NV, sweep(lambda s: s == ustarf, True), cnt)

            base = img * rows_per_img
            for g in range(_KP // 16):
                vec = jnp.zeros((16,), i32)
                for l in range(16):
                    vec = jnp.where(iota16 == l, sm[g * 16 + l], vec)
                idxv[pl.ds(g * 16, 16)] = vec
                rowv[g // 8, pl.ds((g % 8) * 16, 16)] = base + (
                    vec >> 3)

            cps = []
            for c in range(4):
                cps.append(pltpu.async_copy(
                    feats_hbm.at[rowv.at[c]],
                    gat.at[pl.ds(c * 128, 128)], sem))
            for cp in cps:
                cp.wait()

            def repack(g, _):
                idx = idxv[pl.ds(g * 16, 16)]
                slot = idx & (_PACK - 1)
                for l in range(16):
                    k = g * 16 + l
                    ov[pl.ds(k * _FW, _FW)] = gat[
                        k, pl.ds(slot[l] * _FW, _FW)]
                return 0
            lax.fori_loop(0, _KP // 16, repack, 0)

            pltpu.sync_copy(ov, out_hbm.at[img])

    return sel_kernel(scores, feats_packed)


def _dense_body(sel_ref, out_ref, self_ref):
    img = pl.program_id(0)
    f32 = jnp.float32

    @pl.when(img == 0)
    def _init():
        out_ref[...] = jnp.zeros((1, 1), f32)

    sel = sel_ref[0]
    selTv = jnp.transpose(sel, (1, 0))

    si = selTv[0:1, :]
    ii_r = selTv[6:7, :]
    rank = jnp.zeros((1, _KP), f32)
    for jc in range(2):
        sj = lax.slice(sel, (jc * 256, 0), (jc * 256 + 256, 1))
        ij = lax.slice(sel, (jc * 256, 6), (jc * 256 + 256, 7))
        g = (sj > si) | ((sj == si) & (ij < ii_r))
        rank = rank + jnp.sum(g.astype(f32), axis=0, keepdims=True)

    for rb in range(_KP // 128):
        rr = _fiota((128, _KP), 0) + float(rb * 128)
        ptile = jnp.where((rr == rank) & (rank < float(_K)), 1.0, 0.0)
        self_ref[pl.ds(rb * 128, 128), :] = jnp.dot(
            ptile.astype(f32), sel, preferred_element_type=f32)

    selT = jnp.transpose(self_ref[...], (1, 0))

    x1r, y1r = selT[1:2, :], selT[2:3, :]
    x2r, y2r = selT[3:4, :], selT[4:5, :]
    area_r = (x2r - x1r) * (y2r - y1r)
    jjl = _fiota((1, _KP), 1)

    ns_cols = []
    for rb in range(_KP // 128):
        rows = pl.ds(rb * 128, 128)
        x1c, y1c = self_ref[rows, 1:2], self_ref[rows, 2:3]
        x2c, y2c = self_ref[rows, 3:4], self_ref[rows, 4:5]
        area_c = (x2c - x1c) * (y2c - y1c)
        iw = jnp.maximum(jnp.minimum(x2c, x2r) - jnp.maximum(x1c, x1r), 0.0)
        ih = jnp.maximum(jnp.minimum(y2c, y2r) - jnp.maximum(y1c, y1r), 0.0)
        inter = iw * ih
        iou = inter / (area_c + area_r - inter + 1e-9)
        lg = jnp.log(jax.nn.sigmoid((_THR - iou) / _TEMP) + 1e-12)
        iic = _fiota((128, 1), 0) + float(rb * 128)
        lk = jnp.sum(jnp.where(jjl < iic, lg, 0.0), axis=1, keepdims=True)
        ns_cols.append(self_ref[rows, 0:1] * jnp.exp(lk))
    ns = jnp.concatenate(ns_cols, axis=0)
    nsr = jnp.transpose(ns, (1, 0))

    tr = selT[5:6, :]
    valid = jjl < float(_K)
    n_pos = jnp.sum(self_ref[:, 5:6])
    acc_ap = jnp.zeros((1, 1), f32)
    for rb in range(_KP // 128):
        rows = pl.ds(rb * 128, 128)
        s_i = lax.slice(ns, (rb * 128, 0), (rb * 128 + 128, 1))
        sg = jax.nn.sigmoid((nsr - s_i) / _TAU)
        iic = _fiota((128, 1), 0) + float(rb * 128)
        w = jnp.where((jjl != iic) & valid, sg, 0.0)
        rank_all = 1.0 + jnp.sum(w, axis=1, keepdims=True)
        rank_pos = 1.0 + jnp.sum(w * tr, axis=1, keepdims=True)
        prec = rank_pos / rank_all
        acc_ap += jnp.sum(prec * self_ref[rows, 5:6], axis=0,
                          keepdims=True).reshape(1, 1)
    ap = acc_ap / jnp.maximum(n_pos, 1.0)
    loss = jnp.where(n_pos > 0.0, 1.0 - ap, jnp.zeros((1, 1), f32))
    out_ref[...] += loss / float(_B)


def _tc_dense(sel):
    f32 = jnp.float32
    out = pl.pallas_call(
        _dense_body,
        grid=(_B,),
        in_specs=[pl.BlockSpec((1, _KP, _FW), lambda i: (i, 0, 0))],
        out_specs=pl.BlockSpec((1, 1), lambda i: (0, 0)),
        out_shape=jax.ShapeDtypeStruct((1, 1), f32),
        scratch_shapes=[pltpu.VMEM((_KP, _FW), f32)],
        compiler_params=pltpu.CompilerParams(
            dimension_semantics=("arbitrary",)),
    )(sel)
    return out[0, 0]


@jax.jit
def _run(preds, pred, true):
    predsT = jnp.swapaxes(preds, 1, 2)
    trueT = jnp.swapaxes(true, 0, 1)
    scores, feats = _tc_prep(preds, predsT, pred, trueT)
    feats_packed = feats.reshape((_B * _NF * _FW) // 128, 128)
    selflat = _sc_select(scores, feats_packed)
    sel = selflat.reshape(_B, _KP, _FW)
    return _tc_dense(sel)


def kernel(preds, pred, true):
    return _run(preds, pred, true)
